# Initial kernel scaffold; baseline (speedup 1.0000x reference)
import jax
import jax.numpy as jnp
from jax.experimental import pallas as pl


def kernel(input_0, input_1, input_2, availabilities, selection_probabilities, W0, b0, W1, b1, W2, b2):
    raise NotImplementedError("write your pallas kernel here")


# speedup vs baseline: 1.1378x; 1.1378x over previous
import jax
import jax.numpy as jnp
import numpy as np
from jax.experimental import pallas as pl

_B = 16384
_C = 256
_M = 3
_BR = 256

_TINY = np.float32(np.finfo(np.float32).tiny)
_SCALE = np.float32(1.0) - _TINY

_K0 = np.uint32(0)
_K1 = np.uint32(42)
_K2 = _K0 ^ _K1 ^ np.uint32(0x1BD11BDA)
_KS = (_K0, _K1, _K2)
_ROTS = ((13, 15, 26, 6), (17, 29, 16, 24))


def _threefry_bits(j):
    x0 = jnp.zeros_like(j) + _K0
    x1 = j + _K1
    for i in range(5):
        for r in _ROTS[i % 2]:
            x0 = x0 + x1
            x1 = (x1 << np.uint32(r)) | (x1 >> np.uint32(32 - r))
            x1 = x1 ^ x0
        x0 = x0 + _KS[(i + 1) % 3]
        x1 = x1 + _KS[(i + 2) % 3] + np.uint32(i + 1)
    return x0 ^ x1


def _gumbel_from_count(j):
    bits = _threefry_bits(j)
    fb = (bits >> np.uint32(9)) | np.uint32(0x3F800000)
    f = jax.lax.bitcast_convert_type(fb, jnp.float32) - np.float32(1.0)
    u = jnp.maximum(_TINY, f * _SCALE + _TINY)
    return -jnp.log(-jnp.log(u))


def _fused_kernel(x0_ref, x1_ref, x2_ref, sp_ref, av_ref,
                  w0_ref, w1_ref, w2_ref, b_ref, out_ref):
    d0 = jnp.maximum(jnp.dot(x0_ref[...], w0_ref[...],
                             preferred_element_type=jnp.float32)
                     + b_ref[0:1, :], 0.0)
    d1 = jnp.maximum(jnp.dot(x1_ref[...], w1_ref[...],
                             preferred_element_type=jnp.float32)
                     + b_ref[1:2, :], 0.0)
    d2 = jnp.maximum(jnp.dot(x2_ref[...], w2_ref[...],
                             preferred_element_type=jnp.float32)
                     + b_ref[2:3, :], 0.0)

    w = sp_ref[...] * av_ref[...]
    p = w / jnp.sum(w, axis=1, keepdims=True)
    t = jnp.log(p + np.float32(1e-20))

    pid = pl.program_id(0)
    base = pid * (_BR * _C * _M)
    r = jax.lax.broadcasted_iota(jnp.int32, (_BR, _C), 0)
    c = jax.lax.broadcasted_iota(jnp.int32, (_BR, _C), 1)
    q = base + r * (_C * _M) + c * _M

    s0 = t[:, 0:1] + _gumbel_from_count(q.astype(jnp.uint32))
    s1 = t[:, 1:2] + _gumbel_from_count((q + 1).astype(jnp.uint32))
    s2 = t[:, 2:3] + _gumbel_from_count((q + 2).astype(jnp.uint32))

    out_ref[...] = jnp.where(
        (s0 >= s1) & (s0 >= s2), d0, jnp.where(s1 >= s2, d1, d2))


def _run(input_0, input_1, input_2, sp8, av8, w0t, w1t, w2t, bstack,
         interpret=False):
    return pl.pallas_call(
        _fused_kernel,
        grid=(_B // _BR,),
        in_specs=[
            pl.BlockSpec((_BR, 512), lambda i: (i, 0)),
            pl.BlockSpec((_BR, 256), lambda i: (i, 0)),
            pl.BlockSpec((_BR, 128), lambda i: (i, 0)),
            pl.BlockSpec((_BR, 8), lambda i: (i, 0)),
            pl.BlockSpec((_BR, 8), lambda i: (i, 0)),
            pl.BlockSpec((512, 256), lambda i: (0, 0)),
            pl.BlockSpec((256, 256), lambda i: (0, 0)),
            pl.BlockSpec((128, 256), lambda i: (0, 0)),
            pl.BlockSpec((3, 256), lambda i: (0, 0)),
        ],
        out_specs=pl.BlockSpec((_BR, _C), lambda i: (i, 0)),
        out_shape=jax.ShapeDtypeStruct((_B, _C), jnp.float32),
        interpret=interpret,
    )(input_0, input_1, input_2, sp8, av8, w0t, w1t, w2t, bstack)


def kernel(input_0, input_1, input_2, availabilities, selection_probabilities,
           W0, b0, W1, b1, W2, b2):
    pad = jnp.zeros((_B, 8 - _M), jnp.float32)
    sp8 = jnp.concatenate([selection_probabilities.astype(jnp.float32), pad],
                          axis=1)
    av8 = jnp.concatenate([availabilities.astype(jnp.float32), pad], axis=1)
    bstack = jnp.stack([b0, b1, b2], axis=0)
    return _run(input_0, input_1, input_2, sp8, av8,
                W0.T, W1.T, W2.T, bstack)


# speedup vs baseline: 1.1627x; 1.0219x over previous
import jax
import jax.numpy as jnp
import numpy as np
from jax.experimental import pallas as pl

_B = 16384
_C = 256
_M = 3
_BR = 256

_TINY = np.float32(np.finfo(np.float32).tiny)
_SCALE = np.float32(1.0) - _TINY

_K0 = np.uint32(0)
_K1 = np.uint32(42)
_K2 = _K0 ^ _K1 ^ np.uint32(0x1BD11BDA)
_KS = (_K0, _K1, _K2)
_ROTS = ((13, 15, 26, 6), (17, 29, 16, 24))


def _threefry_bits(j):
    x0 = jnp.zeros_like(j) + _K0
    x1 = j + _K1
    for i in range(5):
        for r in _ROTS[i % 2]:
            x0 = x0 + x1
            x1 = (x1 << np.uint32(r)) | (x1 >> np.uint32(32 - r))
            x1 = x1 ^ x0
        x0 = x0 + _KS[(i + 1) % 3]
        x1 = x1 + _KS[(i + 2) % 3] + np.uint32(i + 1)
    return x0 ^ x1


def _neglog_u_from_count(j):
    bits = _threefry_bits(j)
    fb = (bits >> np.uint32(9)) | np.uint32(0x3F800000)
    f = jax.lax.bitcast_convert_type(fb, jnp.float32) - np.float32(1.0)
    u = jnp.maximum(_TINY, f * _SCALE + _TINY)
    return -jnp.log(u)


def _fused_kernel(x0_ref, x1_ref, x2_ref, sp_ref, av_ref,
                  w0_ref, w1_ref, w2_ref, b_ref, out_ref):
    d0 = jnp.maximum(jnp.dot(x0_ref[...], w0_ref[...],
                             preferred_element_type=jnp.float32)
                     + b_ref[0:1, :], 0.0)
    d1 = jnp.maximum(jnp.dot(x1_ref[...], w1_ref[...],
                             preferred_element_type=jnp.float32)
                     + b_ref[1:2, :], 0.0)
    d2 = jnp.maximum(jnp.dot(x2_ref[...], w2_ref[...],
                             preferred_element_type=jnp.float32)
                     + b_ref[2:3, :], 0.0)

    w = sp_ref[...] * av_ref[...]
    p = w / jnp.sum(w, axis=1, keepdims=True) + np.float32(1e-20)

    pid = pl.program_id(0)
    base = pid * (_BR * _C * _M)
    r = jax.lax.broadcasted_iota(jnp.int32, (_BR, _C), 0)
    c = jax.lax.broadcasted_iota(jnp.int32, (_BR, _C), 1)
    q = base + r * (_C * _M) + c * _M

    l0 = _neglog_u_from_count(q.astype(jnp.uint32))
    l1 = _neglog_u_from_count((q + 1).astype(jnp.uint32))
    l2 = _neglog_u_from_count((q + 2).astype(jnp.uint32))
    p0 = p[:, 0:1]
    p1 = p[:, 1:2]
    p2 = p[:, 2:3]

    pick0 = (p0 * l1 >= p1 * l0) & (p0 * l2 >= p2 * l0)
    pick1 = p1 * l2 >= p2 * l1
    out_ref[...] = jnp.where(pick0, d0, jnp.where(pick1, d1, d2))


def _run(input_0, input_1, input_2, sp8, av8, w0t, w1t, w2t, bstack,
         interpret=False):
    return pl.pallas_call(
        _fused_kernel,
        grid=(_B // _BR,),
        in_specs=[
            pl.BlockSpec((_BR, 512), lambda i: (i, 0)),
            pl.BlockSpec((_BR, 256), lambda i: (i, 0)),
            pl.BlockSpec((_BR, 128), lambda i: (i, 0)),
            pl.BlockSpec((_BR, 8), lambda i: (i, 0)),
            pl.BlockSpec((_BR, 8), lambda i: (i, 0)),
            pl.BlockSpec((512, 256), lambda i: (0, 0)),
            pl.BlockSpec((256, 256), lambda i: (0, 0)),
            pl.BlockSpec((128, 256), lambda i: (0, 0)),
            pl.BlockSpec((3, 256), lambda i: (0, 0)),
        ],
        out_specs=pl.BlockSpec((_BR, _C), lambda i: (i, 0)),
        out_shape=jax.ShapeDtypeStruct((_B, _C), jnp.float32),
        interpret=interpret,
    )(input_0, input_1, input_2, sp8, av8, w0t, w1t, w2t, bstack)


def kernel(input_0, input_1, input_2, availabilities, selection_probabilities,
           W0, b0, W1, b1, W2, b2):
    pad = jnp.zeros((_B, 8 - _M), jnp.float32)
    sp8 = jnp.concatenate([selection_probabilities.astype(jnp.float32), pad],
                          axis=1)
    av8 = jnp.concatenate([availabilities.astype(jnp.float32), pad], axis=1)
    bstack = jnp.stack([b0, b1, b2], axis=0)
    return _run(input_0, input_1, input_2, sp8, av8,
                W0.T, W1.T, W2.T, bstack)
